# baseline (speedup 1.0000x reference)
import functools
import math

import jax
import jax.numpy as jnp
from jax import lax
from jax.experimental import pallas as pl
from jax.experimental.pallas import tpu as pltpu

_ALPHA = 0.05
_NC = 4
_CT = 16
_M = 16
_K0 = 8
_K1 = 4
_BBLK = 128
_CHAIN = 32

_NS_PLAN = ((1.998001998001998, 13), (1.3715, 9), (1.2195, 6))
_GFIN = 1.0796
_TAYLOR_N = 18


def _bmm(a, b):
    m = a.shape[0]
    acc = None
    for k in range(m):
        t = a[:, k, :][:, None, :] * b[k, :, :][None, :, :]
        acc = t if acc is None else acc + t
    return acc


def _front_chain(xrh, wt0, wt1, sh0, sh1, wb, *, T):
    Ct = _CT
    Wp = T + 1
    PAD = 128
    hb = xrh.shape[1]
    p0, p1 = _K0 // 2, _K1 // 2
    offsets = sorted({dt - p0 for dt in range(_K0)}
                     | {dt - p1 for dt in range(_K1)})

    zp = jnp.zeros((Ct, hb, PAD), jnp.float32)
    xp = jnp.concatenate([zp, xrh, zp], axis=2)

    h0 = None
    h1 = None
    for s in offsets:
        sl = xp[:, :, PAD + s:PAD + s + Wp]
        dt0 = s + p0
        if 0 <= dt0 < _K0:
            t = wt0[:, dt0][:, None, None] * sl
            h0 = t if h0 is None else h0 + t
        dt1 = s + p1
        if 0 <= dt1 < _K1:
            t = wt1[:, dt1][:, None, None] * sl
            h1 = t if h1 is None else h1 + t
    h0 = h0 + sh0
    h1 = h1 + sh1

    g = (lax.dot_general(wb[:Ct], h0, (((0,), (0,)), ((), ())),
                         preferred_element_type=jnp.float32)
         + lax.dot_general(wb[Ct:], h1, (((0,), (0,)), ((), ())),
                           preferred_element_type=jnp.float32))

    ssk = jnp.sum(h0 * h0 + h1 * h1, axis=0)
    ss = jnp.sum(ssk, axis=1)

    gram = lax.dot_general(g, g, (((2,), (2,)), ((1,), (1,))),
                           preferred_element_type=jnp.float32)
    return gram, ss


def _logm_fc(a, wcls, bias):
    m = _M
    Bg = a.shape[2]
    row = lax.broadcasted_iota(jnp.int32, (m, m, 1), 0)
    col = lax.broadcasted_iota(jnp.int32, (m, m, 1), 1)
    eye = (row == col).astype(jnp.float32)
    eyeb = jnp.broadcast_to(eye, (m, m, Bg))

    tr = jnp.sum(a * eye, axis=(0, 1), keepdims=True)
    tr = jnp.maximum(tr, 1e-30)
    acur = a * (1.0 / tr)

    for c, iters in _NS_PLAN:
        y0 = c * acur
        y = 1.5 * y0 - 0.5 * _bmm(y0, y0)
        z = 1.5 * eyeb - 0.5 * y0
        for it in range(1, iters):
            w = _bmm(z, y)
            y = 1.5 * y - 0.5 * _bmm(y, w)
            if it < iters - 1:
                z = 1.5 * z - 0.5 * _bmm(w, z)
        acur = y

    x = eyeb - _GFIN * acur
    x2 = _bmm(x, x)
    x3 = _bmm(x2, x)
    x4 = _bmm(x3, x)
    pows = (eyeb, x, x2, x3)
    ngrp = (_TAYLOR_N + 3) // 4
    acc = None
    for i in range(ngrp - 1, -1, -1):
        grp = None
        for r in range(4):
            j = 4 * i + r
            if j < _TAYLOR_N:
                term = (1.0 / (j + 1)) * pows[r]
                grp = term if grp is None else grp + term
        acc = grp if acc is None else grp + _bmm(x4, acc)
    lg = -_bmm(x, acc)
    s = len(_NS_PLAN)
    kconst = -float(2 ** s) * math.log(_GFIN)
    for i, (c, _) in enumerate(_NS_PLAN):
        kconst -= float(2 ** i) * math.log(c)
    logm = float(2 ** s) * lg + (jnp.log(tr) + kconst) * eye

    rows = [jnp.sum(logm * wcls[n][:, :, None], axis=(0, 1))[None, :]
            for n in range(_NC)]
    rows.append(jnp.zeros((8 - _NC, Bg), jnp.float32))
    out = jnp.concatenate(rows, axis=0) + bias
    return jnp.transpose(out)


def _fused_body(x_ref, arep_ref, crep_ref, wt0_ref, wt1_ref, sh0_ref,
                sh1_ref, wb_ref, wtw_ref, wcls_ref, bias_ref, out_ref,
                *, T, c_gram, c_mu):
    Ct, m, Bblk = _CT, _M, _BBLK

    xr = lax.dot_general(arep_ref[...], x_ref[...], (((1,), (1,)), ((), ())),
                         preferred_element_type=jnp.float32)
    xr = xr + crep_ref[...][:, :, None]

    wt0 = wt0_ref[...]
    wt1 = wt1_ref[...]
    wb = wb_ref[...]
    sh0 = sh0_ref[...][:, :, None]
    sh1 = sh1_ref[...][:, :, None]

    grams = []
    sss = []
    for c0 in range(0, Bblk, _CHAIN):
        gram, ss = _front_chain(xr[:, c0:c0 + _CHAIN, :],
                                wt0, wt1, sh0, sh1, wb, T=T)
        grams.append(gram)
        sss.append(ss)

    gram = jnp.concatenate(grams, axis=0)
    ss = jnp.concatenate(sss, axis=0)
    mapped = (c_gram * gram
              + (c_mu * ss)[:, None, None] * wtw_ref[...][None])

    a = jnp.transpose(mapped, (1, 2, 0))
    out_ref[...] = _logm_fc(a, wcls_ref[...],
                            bias_ref[...]).astype(out_ref.dtype)


def kernel(x, Arep, crep, wt_0, wt_1, sh_0, sh_1, Wb, WtW, Wcls, bias_pad):
    B, C, T = x.shape
    m = _M
    denom = float(T - 1)
    c_gram = (1.0 - _ALPHA) / denom
    c_mu = _ALPHA / (float(Wb.shape[0]) * denom)
    bblk = _BBLK
    grid = (B // bblk,)
    bias_col = jnp.transpose(bias_pad)[:8]

    body = functools.partial(_fused_body, T=T, c_gram=c_gram, c_mu=c_mu)
    out = pl.pallas_call(
        body,
        out_shape=jax.ShapeDtypeStruct((B, 8), jnp.float32),
        grid=grid,
        in_specs=[
            pl.BlockSpec((bblk, C, T), lambda g: (g, 0, 0)),
            pl.BlockSpec((_CT, C), lambda g: (0, 0)),
            pl.BlockSpec((_CT, 1), lambda g: (0, 0)),
            pl.BlockSpec((_CT, _K0), lambda g: (0, 0)),
            pl.BlockSpec((_CT, _K1), lambda g: (0, 0)),
            pl.BlockSpec((_CT, 1), lambda g: (0, 0)),
            pl.BlockSpec((_CT, 1), lambda g: (0, 0)),
            pl.BlockSpec((2 * _CT, m), lambda g: (0, 0)),
            pl.BlockSpec((m, m), lambda g: (0, 0)),
            pl.BlockSpec((_NC, m, m), lambda g: (0, 0, 0)),
            pl.BlockSpec((8, 1), lambda g: (0, 0)),
        ],
        out_specs=pl.BlockSpec((bblk, 8), lambda g: (g, 0)),
        compiler_params=pltpu.CompilerParams(
            dimension_semantics=("parallel",)),
    )(x, Arep, crep, wt_0, wt_1, sh_0, sh_1, Wb, WtW, Wcls, bias_col)
    return out[:, :_NC]


# speedup vs baseline: 1.3795x; 1.3795x over previous
import functools
import math

import jax
import jax.numpy as jnp
from jax import lax
from jax.experimental import pallas as pl
from jax.experimental.pallas import tpu as pltpu

_BN_EPS = 1e-5
_ALPHA = 0.05
_NC = 4
_CT = 16
_M = 16
_K0 = 8
_K1 = 4
_BBLK = 64


def _front_body(x_ref, arep_ref, crep_ref, wt0_ref, wt1_ref, sh0_ref,
                sh1_ref, wb_ref, wtw_ref, out_ref, *, T, c_gram, c_mu):
    Ct, m, Bblk = _CT, _M, _BBLK
    Wp = T + 1
    PAD = 128

    xr = lax.dot_general(arep_ref[...], x_ref[...], (((1,), (1,)), ((), ())),
                         preferred_element_type=jnp.float32)
    xr = xr + crep_ref[...][:, :, None]

    wt0 = wt0_ref[...]
    wt1 = wt1_ref[...]
    wb = wb_ref[...]
    sh0 = sh0_ref[...][:, :, None]
    sh1 = sh1_ref[...][:, :, None]
    p0, p1 = _K0 // 2, _K1 // 2
    offsets = sorted({dt - p0 for dt in range(_K0)}
                     | {dt - p1 for dt in range(_K1)})

    hb = Bblk // 2
    zp = jnp.zeros((Ct, hb, PAD), jnp.float32)
    grams = []
    sss = []
    for half in range(2):
        xrh = xr[:, half * hb:(half + 1) * hb, :]
        xp = jnp.concatenate([zp, xrh, zp], axis=2)

        h0 = None
        h1 = None
        for s in offsets:
            sl = xp[:, :, PAD + s:PAD + s + Wp]
            dt0 = s + p0
            if 0 <= dt0 < _K0:
                t = wt0[:, dt0][:, None, None] * sl
                h0 = t if h0 is None else h0 + t
            dt1 = s + p1
            if 0 <= dt1 < _K1:
                t = wt1[:, dt1][:, None, None] * sl
                h1 = t if h1 is None else h1 + t
        h0 = h0 + sh0
        h1 = h1 + sh1

        g = (lax.dot_general(wb[:Ct], h0, (((0,), (0,)), ((), ())),
                             preferred_element_type=jnp.float32)
             + lax.dot_general(wb[Ct:], h1, (((0,), (0,)), ((), ())),
                               preferred_element_type=jnp.float32))

        ssk = jnp.sum(h0 * h0 + h1 * h1, axis=0)
        sss.append(jnp.sum(ssk, axis=1))

        grams.append(lax.dot_general(g, g, (((2,), (2,)), ((1,), (1,))),
                                     preferred_element_type=jnp.float32))

    gram = jnp.concatenate(grams, axis=0)
    ss = jnp.concatenate(sss, axis=0)
    out = c_gram * gram + (c_mu * ss)[:, None, None] * wtw_ref[...][None]
    out_ref[...] = out.astype(out_ref.dtype)


_NS_PLAN = ((1.998001998001998, 12), (1.3715, 8), (1.2195, 5))
_GFIN = 1.0796
_TAYLOR_N = 15


def _bmm(a, b):
    m = a.shape[0]
    acc = None
    for k in range(m):
        t = a[:, k, :][:, None, :] * b[k, :, :][None, :, :]
        acc = t if acc is None else acc + t
    return acc


def _logm_fc_body(a_ref, wcls_ref, bias_ref, out_ref):
    m = _M
    a = jnp.transpose(a_ref[...], (1, 2, 0))
    Bg = a.shape[2]
    row = lax.broadcasted_iota(jnp.int32, (m, m, 1), 0)
    col = lax.broadcasted_iota(jnp.int32, (m, m, 1), 1)
    eye = (row == col).astype(jnp.float32)
    eyeb = jnp.broadcast_to(eye, (m, m, Bg))

    tr = jnp.sum(a * eye, axis=(0, 1), keepdims=True)
    tr = jnp.maximum(tr, 1e-30)
    acur = a * (1.0 / tr)

    for c, iters in _NS_PLAN:
        y0 = c * acur
        yy = _bmm(y0, y0)
        y = 1.5 * y0 - 0.5 * yy
        z = 1.5 * eyeb - 0.5 * y0
        for it in range(1, iters):
            w = _bmm(z, y)
            y = 1.5 * y - 0.5 * _bmm(y, w)
            if it < iters - 1:
                z = 1.5 * z - 0.5 * _bmm(w, z)
        acur = y

    x = eyeb - _GFIN * acur
    x2 = _bmm(x, x)
    x3 = _bmm(x2, x)
    x4 = _bmm(x3, x)
    pows = (eyeb, x, x2, x3)
    ngrp = (_TAYLOR_N + 3) // 4
    acc = None
    for i in range(ngrp - 1, -1, -1):
        grp = None
        for r in range(4):
            j = 4 * i + r
            if j < _TAYLOR_N:
                term = (1.0 / (j + 1)) * pows[r]
                grp = term if grp is None else grp + term
        acc = grp if acc is None else grp + _bmm(x4, acc)
    lg = -_bmm(x, acc)
    s = len(_NS_PLAN)
    kconst = -float(2 ** s) * math.log(_GFIN)
    for i, (c, _) in enumerate(_NS_PLAN):
        kconst -= float(2 ** i) * math.log(c)
    logm = float(2 ** s) * lg + (jnp.log(tr) + kconst) * eye

    wcls = wcls_ref[...]
    rows = [jnp.sum(logm * wcls[n][:, :, None], axis=(0, 1))[None, :]
            for n in range(_NC)]
    rows.append(jnp.zeros((8 - _NC, Bg), jnp.float32))
    out = jnp.concatenate(rows, axis=0) + bias_ref[...]
    out_ref[...] = jnp.transpose(out).astype(out_ref.dtype)


def kernel(x, Arep, crep, wt_0, wt_1, sh_0, sh_1, Wb, WtW, Wcls, bias_pad):
    B, C, T = x.shape
    m = _M
    denom = float(T - 1)
    c_gram = (1.0 - _ALPHA) / denom
    c_mu = _ALPHA / (float(Wb.shape[0]) * denom)
    bblk = _BBLK
    grid = (B // bblk,)

    body = functools.partial(_front_body, T=T, c_gram=c_gram, c_mu=c_mu)
    mapped = pl.pallas_call(
        body,
        out_shape=jax.ShapeDtypeStruct((B, m, m), jnp.float32),
        grid=grid,
        in_specs=[
            pl.BlockSpec((bblk, C, T), lambda g: (g, 0, 0)),
            pl.BlockSpec((_CT, C), lambda g: (0, 0)),
            pl.BlockSpec((_CT, 1), lambda g: (0, 0)),
            pl.BlockSpec((_CT, _K0), lambda g: (0, 0)),
            pl.BlockSpec((_CT, _K1), lambda g: (0, 0)),
            pl.BlockSpec((_CT, 1), lambda g: (0, 0)),
            pl.BlockSpec((_CT, 1), lambda g: (0, 0)),
            pl.BlockSpec((2 * _CT, m), lambda g: (0, 0)),
            pl.BlockSpec((m, m), lambda g: (0, 0)),
        ],
        out_specs=pl.BlockSpec((bblk, m, m), lambda g: (g, 0, 0)),
        compiler_params=pltpu.CompilerParams(
            dimension_semantics=("parallel",)),
    )(x, Arep, crep, wt_0, wt_1, sh_0, sh_1, Wb, WtW)

    bias_col = jnp.transpose(bias_pad)[:8]
    bg = B // 2
    out = pl.pallas_call(
        _logm_fc_body,
        out_shape=jax.ShapeDtypeStruct((B, 8), jnp.float32),
        grid=(B // bg,),
        in_specs=[
            pl.BlockSpec((bg, m, m), lambda g: (g, 0, 0)),
            pl.BlockSpec((_NC, m, m), lambda g: (0, 0, 0)),
            pl.BlockSpec((8, 1), lambda g: (0, 0)),
        ],
        out_specs=pl.BlockSpec((bg, 8), lambda g: (g, 0)),
        compiler_params=pltpu.CompilerParams(
            dimension_semantics=("parallel",)),
    )(mapped, Wcls, bias_col)
    return out[:, :_NC]
